# baseline (speedup 1.0000x reference)
import functools

import jax
import jax.numpy as jnp
from jax.experimental import pallas as pl


def _elu(x):
    return jnp.where(x > 0, x, jnp.exp(jnp.minimum(x, 0.0)) - 1.0)


def _head_body(pooled_ref, f1w_ref, f1b_ref, f2w_ref, f2b_ref, out_ref):
    pooled = pooled_ref[...]
    h = _elu(pooled @ f1w_ref[...].T + f1b_ref[...][None, :])
    logits = h @ f2w_ref[...].T + f2b_ref[...][None, :]
    m = jnp.max(logits, axis=-1, keepdims=True)
    s = logits - m
    lse = jnp.log(jnp.sum(jnp.exp(s), axis=-1, keepdims=True))
    out_ref[...] = s - lse


def _head(pooled, f1w, f1b, f2w, f2b):
    return pl.pallas_call(
        _head_body,
        out_shape=jax.ShapeDtypeStruct((pooled.shape[0], f2w.shape[0]),
                                       jnp.float32),
    )(pooled, f1w, f1b, f2w, f2b)


def _ggc(x, src, dst, W, wih, whh, bih, bhh):
    out_ch = W.shape[-1]
    n, in_ch = x.shape
    if in_ch < out_ch:
        x = jnp.concatenate(
            [x, jnp.zeros((n, out_ch - in_ch), dtype=x.dtype)], axis=1)
    h = x
    for i in range(W.shape[0]):
        m = h @ W[i]
        agg = jnp.zeros((n, out_ch), dtype=h.dtype).at[dst].add(m[src])
        gi = agg @ wih.T + bih
        gh = h @ whh.T + bhh
        ir, iz, i_n = jnp.split(gi, 3, axis=1)
        hr, hz, h_n = jnp.split(gh, 3, axis=1)
        r = jax.nn.sigmoid(ir + hr)
        z = jax.nn.sigmoid(iz + hz)
        nn = jnp.tanh(i_n + r * h_n)
        h = (1.0 - z) * nn + z * h
    return h


def _bn(x, g, b, m, v):
    return (x - m) / jnp.sqrt(v + 1e-5) * g + b


def kernel(x, edge_index, batch, conv1_w, gru1_wih, gru1_whh, gru1_bih,
           gru1_bhh, bn1_g, bn1_b, bn1_m, bn1_v, conv2_w, gru2_wih, gru2_whh,
           gru2_bih, gru2_bhh, bn2_g, bn2_b, bn2_m, bn2_v, conv3_w, gru3_wih,
           gru3_whh, gru3_bih, gru3_bhh, fc1_w, fc1_b, fc2_w, fc2_b):
    src = edge_index[0]
    dst = edge_index[1]
    h = _elu(_ggc(x, src, dst, conv1_w, gru1_wih, gru1_whh, gru1_bih,
                  gru1_bhh))
    h = _bn(h, bn1_g, bn1_b, bn1_m, bn1_v)
    h = _elu(_ggc(h, src, dst, conv2_w, gru2_wih, gru2_whh, gru2_bih,
                  gru2_bhh))
    h = _bn(h, bn2_g, bn2_b, bn2_m, bn2_v)
    h = _elu(_ggc(h, src, dst, conv3_w, gru3_wih, gru3_whh, gru3_bih,
                  gru3_bhh))
    pooled = jax.ops.segment_sum(h, batch, num_segments=128)
    return _head(pooled, fc1_w, fc1_b, fc2_w, fc2_b)


# speedup vs baseline: 3.6031x; 3.6031x over previous
import functools

import jax
import jax.numpy as jnp
from jax import lax
from jax.experimental import pallas as pl
from jax.experimental.pallas import tpu as pltpu
from jax.experimental.pallas import tpu_sc as plsc

N = 50000
E = 800000
NG = 128

_NC = 2
_NS = 16
_KB = 128
_BLK_PER_CHUNK = 8
_CHUNK = _KB * _BLK_PER_CHUNK
_EDGE_BLOCKS = -(-E // (_KB * _NS * _BLK_PER_CHUNK)) * _NS * _BLK_PER_CHUNK
_EPAD = _EDGE_BLOCKS * _KB
_BLK_PER_SUB = _EDGE_BLOCKS // _NS
_CHUNKS_PER_SUB = _BLK_PER_SUB // _BLK_PER_CHUNK
_RPS = 3128
_RPS_BASE = 3080
_RPS_EXTRA = _RPS - _RPS_BASE
_ZROWS = 616
_ACC_ROWS = N + 16
_FW = 16


def _expm1(x):
    u = jnp.exp(x)
    em = jnp.where(u == 1.0, x,
                   jnp.where(u == 0.0, -1.0,
                             (u - 1.0) * x / jnp.log(jnp.maximum(u, 1e-38))))
    return em


def _elu(x):
    return jnp.where(x > 0, x, _expm1(jnp.minimum(x, 0.0)))


def _make_edge_agg(S):
    P = S // _NC
    mesh = plsc.VectorSubcoreMesh(core_axis_name="c", subcore_axis_name="s")

    @functools.partial(
        pl.kernel,
        mesh=mesh,
        out_type=jax.ShapeDtypeStruct((S, N, _FW), jnp.float32),
        scratch_types=[
            pltpu.VMEM((_ZROWS, _FW), jnp.float32),
            pltpu.VMEM((_BLK_PER_CHUNK, _KB), jnp.int32),
            pltpu.VMEM((_BLK_PER_CHUNK, _KB), jnp.int32),
            pltpu.VMEM((_BLK_PER_CHUNK, _KB), jnp.int32),
            pltpu.VMEM((_BLK_PER_CHUNK, _KB, _FW), jnp.float32),
            pltpu.VMEM_SHARED((_ACC_ROWS, _FW), jnp.float32),
            pltpu.SemaphoreType.DMA,
            pltpu.SemaphoreType.DMA,
        ],
        compiler_params=pltpu.CompilerParams(use_tc_tiling_on_sc=False),
    )
    def edge_agg(m_hbm, src_hbm, dst_hbm, out_hbm, zbuf, srcv, dstv, gidx,
                 rows, acc, gsem, ssem):
        cid = lax.axis_index("c")
        sid = lax.axis_index("s")

        z16 = jnp.zeros((16,), jnp.float32)

        def zbody(r, _):
            zbuf[r, pl.ds(0, 16)] = z16
            return 0

        lax.fori_loop(0, _ZROWS, zbody, 0)

        r0 = sid * _RPS
        for p in range(P):
            sl = cid * P + p
            off = sl * N

            for zi in range(_RPS_BASE // _ZROWS):
                pltpu.sync_copy(zbuf, acc.at[pl.ds(r0 + zi * _ZROWS, _ZROWS)])

            @pl.when(sid < _NS - 1)
            def _():
                pltpu.sync_copy(
                    zbuf.at[pl.ds(0, _RPS_EXTRA)],
                    acc.at[pl.ds(r0 + _RPS_BASE, _RPS_EXTRA)])

            plsc.subcore_barrier()

            def chunk_body(ch, _):
                blk0 = sid * _BLK_PER_SUB + ch * _BLK_PER_CHUNK
                pltpu.sync_copy(src_hbm.at[pl.ds(blk0, _BLK_PER_CHUNK)], srcv)
                pltpu.sync_copy(dst_hbm.at[pl.ds(blk0, _BLK_PER_CHUNK)], dstv)

                def add_body(r, _):
                    for t in range(_KB // 16):
                        gidx[r, pl.ds(t * 16, 16)] = (
                            srcv[r, pl.ds(t * 16, 16)] + off)
                    return 0

                lax.fori_loop(0, _BLK_PER_CHUNK, add_body, 0)

                copies = []
                for b in range(_BLK_PER_CHUNK):
                    copies.append(pltpu.async_copy(
                        m_hbm.at[gidx.at[b]], rows.at[b], gsem))
                for cp in copies:
                    cp.wait()

                for b in range(_BLK_PER_CHUNK):
                    pltpu.sync_copy(rows.at[b], acc.at[dstv.at[b]], add=True)
                return 0

            lax.fori_loop(0, _CHUNKS_PER_SUB, chunk_body, 0)
            plsc.subcore_barrier()

            pltpu.sync_copy(acc.at[pl.ds(r0, _RPS_BASE)],
                            out_hbm.at[sl, pl.ds(r0, _RPS_BASE)])

            @pl.when(sid < _NS - 1)
            def _():
                pltpu.sync_copy(
                    acc.at[pl.ds(r0 + _RPS_BASE, _RPS_EXTRA)],
                    out_hbm.at[sl, pl.ds(r0 + _RPS_BASE, _RPS_EXTRA)])

            if p + 1 < P:
                plsc.subcore_barrier()

    return edge_agg


def _edge_agg_4(m_flat, src2d, dst2d):
    return _make_edge_agg(4)(m_flat, src2d, dst2d)


def _edge_agg_8(m_flat, src2d, dst2d):
    return _make_edge_agg(8)(m_flat, src2d, dst2d)


def _msplit_body(S, h_ref, w_ref, out_ref):
    h = h_ref[...]
    for s in range(S):
        out_ref[s] = jnp.dot(h, w_ref[s], preferred_element_type=jnp.float32)


def _msplit(h, w_sliced, BN=2000):
    F = h.shape[1]
    S = F // _FW
    nblk = N // BN
    return pl.pallas_call(
        functools.partial(_msplit_body, S),
        grid=(nblk,),
        in_specs=[
            pl.BlockSpec((BN, F), lambda i: (i, 0)),
            pl.BlockSpec((S, F, _FW), lambda i: (0, 0, 0)),
        ],
        out_specs=pl.BlockSpec((S, BN, _FW), lambda i: (0, i, 0)),
        out_shape=jax.ShapeDtypeStruct((S, N, _FW), jnp.float32),
    )(h, w_sliced)


def _gru_body(S, mode, h_ref, agg_ref, wih_ref, whh_ref, bih_ref, bhh_ref,
              bng_ref, bnb_ref, bnm_ref, bnv_ref, out_ref):
    h = h_ref[...]
    aggf = jnp.concatenate([agg_ref[s] for s in range(S)], axis=-1)
    g = []
    for k in range(3):
        gi = jnp.dot(aggf, wih_ref[k], preferred_element_type=jnp.float32)
        gh = jnp.dot(h, whh_ref[k], preferred_element_type=jnp.float32)
        g.append((gi + bih_ref[k][None, :], gh + bhh_ref[k][None, :]))
    r = jax.nn.sigmoid(g[0][0] + g[0][1])
    z = jax.nn.sigmoid(g[1][0] + g[1][1])
    nn = jnp.tanh(g[2][0] + r * g[2][1])
    hn = (1.0 - z) * nn + z * h
    if mode >= 1:
        hn = _elu(hn)
    if mode == 1:
        hn = ((hn - bnm_ref[0][None, :])
              / jnp.sqrt(bnv_ref[0][None, :] + 1e-5)
              * bng_ref[0][None, :] + bnb_ref[0][None, :])
    out_ref[...] = hn


def _gru(h, agg, wihT3, whhT3, bih3, bhh3, bng, bnb, bnm, bnv, mode,
         BN=2000):
    F = h.shape[1]
    S = F // _FW
    nblk = N // BN
    return pl.pallas_call(
        functools.partial(_gru_body, S, mode),
        grid=(nblk,),
        in_specs=[
            pl.BlockSpec((BN, F), lambda i: (i, 0)),
            pl.BlockSpec((S, BN, _FW), lambda i: (0, i, 0)),
            pl.BlockSpec((3, F, F), lambda i: (0, 0, 0)),
            pl.BlockSpec((3, F, F), lambda i: (0, 0, 0)),
            pl.BlockSpec((3, F), lambda i: (0, 0)),
            pl.BlockSpec((3, F), lambda i: (0, 0)),
            pl.BlockSpec((1, F), lambda i: (0, 0)),
            pl.BlockSpec((1, F), lambda i: (0, 0)),
            pl.BlockSpec((1, F), lambda i: (0, 0)),
            pl.BlockSpec((1, F), lambda i: (0, 0)),
        ],
        out_specs=pl.BlockSpec((BN, F), lambda i: (i, 0)),
        out_shape=jax.ShapeDtypeStruct((N, F), jnp.float32),
    )(h, agg, wihT3, whhT3, bih3, bhh3, bng, bnb, bnm, bnv)


def _pool_body(batch_ref, h_ref, out_ref):
    i = pl.program_id(0)

    @pl.when(i == 0)
    def _():
        out_ref[...] = jnp.zeros_like(out_ref)

    b = batch_ref[...]
    seg = jax.lax.broadcasted_iota(jnp.int32, (b.shape[0], NG), 1)
    onehot = (b == seg).astype(jnp.float32)
    out_ref[...] += jax.lax.dot_general(
        onehot, h_ref[...], (((0,), (0,)), ((), ())),
        preferred_element_type=jnp.float32,
        precision=jax.lax.Precision.HIGHEST)


def _pool(batch2d, h, BN=2000):
    F = h.shape[1]
    nblk = N // BN
    return pl.pallas_call(
        _pool_body,
        grid=(nblk,),
        in_specs=[
            pl.BlockSpec((BN, 1), lambda i: (i, 0)),
            pl.BlockSpec((BN, F), lambda i: (i, 0)),
        ],
        out_specs=pl.BlockSpec((NG, F), lambda i: (0, 0)),
        out_shape=jax.ShapeDtypeStruct((NG, F), jnp.float32),
        compiler_params=pltpu.CompilerParams(
            dimension_semantics=("arbitrary",)),
    )(batch2d, h)


def _head_body(pooled_ref, f1w_ref, f1b_ref, f2w_ref, f2b_ref, out_ref):
    pooled = pooled_ref[...]
    h = _elu(jnp.dot(pooled, f1w_ref[...].T,
                     preferred_element_type=jnp.float32) + f1b_ref[...])
    logits = jnp.dot(h, f2w_ref[...].T,
                     preferred_element_type=jnp.float32) + f2b_ref[...]
    m = jnp.max(logits, axis=-1, keepdims=True)
    s = logits - m
    lse = jnp.log(jnp.sum(jnp.exp(s), axis=-1, keepdims=True))
    out_ref[...] = s - lse


def _head(pooled, f1w, f1b, f2w, f2b):
    return pl.pallas_call(
        _head_body,
        out_shape=jax.ShapeDtypeStruct((pooled.shape[0], f2w.shape[0]),
                                       jnp.float32),
    )(pooled, f1w, f1b.reshape(1, -1), f2w, f2b.reshape(1, -1))


def _layer(h, src2d, dst2d, conv_w, wih, whh, bih, bhh, bn_params,
           final_mode):
    F = conv_w.shape[-1]
    S = F // _FW
    steps = conv_w.shape[0]
    w_sliced = conv_w.reshape(steps, F, S, _FW).transpose(0, 2, 1, 3)
    wihT3 = wih.reshape(3, F, F).transpose(0, 2, 1)
    whhT3 = whh.reshape(3, F, F).transpose(0, 2, 1)
    bih3 = bih.reshape(3, F)
    bhh3 = bhh.reshape(3, F)
    edge_agg = _edge_agg_4 if S == 4 else _edge_agg_8
    for i in range(steps):
        m_split = _msplit(h, w_sliced[i])
        agg = edge_agg(m_split.reshape(S * N, _FW), src2d, dst2d)
        mode = final_mode if i == steps - 1 else 0
        h = _gru(h, agg, wihT3, whhT3, bih3, bhh3, *bn_params, mode)
    return h


def kernel(x, edge_index, batch, conv1_w, gru1_wih, gru1_whh, gru1_bih,
           gru1_bhh, bn1_g, bn1_b, bn1_m, bn1_v, conv2_w, gru2_wih, gru2_whh,
           gru2_bih, gru2_bhh, bn2_g, bn2_b, bn2_m, bn2_v, conv3_w, gru3_wih,
           gru3_whh, gru3_bih, gru3_bhh, fc1_w, fc1_b, fc2_w, fc2_b):
    src = edge_index[0]
    dst = edge_index[1]
    npad = _EPAD - E
    src2d = jnp.concatenate(
        [src, jnp.zeros((npad,), jnp.int32)]).reshape(_EDGE_BLOCKS, _KB)
    dst2d = jnp.concatenate(
        [dst, jnp.full((npad,), N, jnp.int32)]).reshape(_EDGE_BLOCKS, _KB)

    bn1 = tuple(a.reshape(1, -1) for a in (bn1_g, bn1_b, bn1_m, bn1_v))
    bn2 = tuple(a.reshape(1, -1) for a in (bn2_g, bn2_b, bn2_m, bn2_v))
    d128 = jnp.zeros((1, 128), jnp.float32)
    bn3 = (d128, d128, d128, d128)

    h = _layer(x, src2d, dst2d, conv1_w, gru1_wih, gru1_whh, gru1_bih,
               gru1_bhh, bn1, final_mode=1)
    h = _layer(h, src2d, dst2d, conv2_w, gru2_wih, gru2_whh, gru2_bih,
               gru2_bhh, bn2, final_mode=1)
    h = jnp.concatenate([h, jnp.zeros((N, 64), jnp.float32)], axis=1)
    h = _layer(h, src2d, dst2d, conv3_w, gru3_wih, gru3_whh, gru3_bih,
               gru3_bhh, bn3, final_mode=2)

    pooled = _pool(batch.reshape(N, 1), h)
    return _head(pooled, fc1_w, fc1_b, fc2_w, fc2_b)


# speedup vs baseline: 3.8434x; 1.0667x over previous
import functools

import jax
import jax.numpy as jnp
from jax import lax
from jax.experimental import pallas as pl
from jax.experimental.pallas import tpu as pltpu
from jax.experimental.pallas import tpu_sc as plsc

N = 50000
E = 800000
NG = 128

_NC = 2
_NS = 16
_KB = 128
_BLK_PER_CHUNK = 8
_CHUNK = _KB * _BLK_PER_CHUNK
_EDGE_BLOCKS = -(-E // (_KB * _NS * _BLK_PER_CHUNK)) * _NS * _BLK_PER_CHUNK
_EPAD = _EDGE_BLOCKS * _KB
_BLK_PER_SUB = _EDGE_BLOCKS // _NS
_CHUNKS_PER_SUB = _BLK_PER_SUB // _BLK_PER_CHUNK
_RPS = 3128
_RPS_BASE = 3080
_RPS_EXTRA = _RPS - _RPS_BASE
_ZROWS = 616
_ACC_ROWS = N + 16
_FW = 16


def _expm1(x):
    u = jnp.exp(x)
    em = jnp.where(u == 1.0, x,
                   jnp.where(u == 0.0, -1.0,
                             (u - 1.0) * x / jnp.log(jnp.maximum(u, 1e-38))))
    return em


def _elu(x):
    return jnp.where(x > 0, x, _expm1(jnp.minimum(x, 0.0)))


def _make_edge_agg(S):
    P = S // _NC
    mesh = plsc.VectorSubcoreMesh(core_axis_name="c", subcore_axis_name="s")

    @functools.partial(
        pl.kernel,
        mesh=mesh,
        out_type=jax.ShapeDtypeStruct((S, N, _FW), jnp.float32),
        scratch_types=[
            pltpu.VMEM((_ZROWS, _FW), jnp.float32),
            pltpu.VMEM((_BLK_PER_CHUNK, _KB), jnp.int32),
            pltpu.VMEM((_BLK_PER_CHUNK, _KB), jnp.int32),
            pltpu.VMEM((_BLK_PER_CHUNK, _KB), jnp.int32),
            pltpu.VMEM((_BLK_PER_CHUNK, _KB, _FW), jnp.float32),
            pltpu.VMEM_SHARED((_ACC_ROWS, _FW), jnp.float32),
            pltpu.SemaphoreType.DMA,
            pltpu.SemaphoreType.DMA,
        ],
        compiler_params=pltpu.CompilerParams(use_tc_tiling_on_sc=False),
    )
    def edge_agg(m_hbm, src_hbm, dst_hbm, out_hbm, zbuf, srcv, dstv, gidx,
                 rows, acc, gsem, ssem):
        cid = lax.axis_index("c")
        sid = lax.axis_index("s")

        z16 = jnp.zeros((16,), jnp.float32)

        def zbody(r, _):
            zbuf[r, pl.ds(0, 16)] = z16
            return 0

        lax.fori_loop(0, _ZROWS, zbody, 0)

        r0 = sid * _RPS
        for p in range(P):
            sl = cid * P + p
            off = sl * N

            for zi in range(_RPS_BASE // _ZROWS):
                pltpu.sync_copy(zbuf, acc.at[pl.ds(r0 + zi * _ZROWS, _ZROWS)])

            @pl.when(sid < _NS - 1)
            def _():
                pltpu.sync_copy(
                    zbuf.at[pl.ds(0, _RPS_EXTRA)],
                    acc.at[pl.ds(r0 + _RPS_BASE, _RPS_EXTRA)])

            plsc.subcore_barrier()

            def chunk_body(ch, _):
                blk0 = sid * _BLK_PER_SUB + ch * _BLK_PER_CHUNK
                pltpu.sync_copy(src_hbm.at[pl.ds(blk0, _BLK_PER_CHUNK)], srcv)
                pltpu.sync_copy(dst_hbm.at[pl.ds(blk0, _BLK_PER_CHUNK)], dstv)

                def add_body(r, _):
                    for t in range(_KB // 16):
                        gidx[r, pl.ds(t * 16, 16)] = (
                            srcv[r, pl.ds(t * 16, 16)] + off)
                    return 0

                lax.fori_loop(0, _BLK_PER_CHUNK, add_body, 0)

                copies = []
                for b in range(_BLK_PER_CHUNK):
                    copies.append(pltpu.async_copy(
                        m_hbm.at[gidx.at[b]], rows.at[b], gsem))
                for cp in copies:
                    cp.wait()

                scopies = []
                for b in range(_BLK_PER_CHUNK):
                    scopies.append(pltpu.async_copy(
                        rows.at[b], acc.at[dstv.at[b]], ssem, add=True))
                for cp in scopies:
                    cp.wait()
                return 0

            lax.fori_loop(0, _CHUNKS_PER_SUB, chunk_body, 0)
            plsc.subcore_barrier()

            pltpu.sync_copy(acc.at[pl.ds(r0, _RPS_BASE)],
                            out_hbm.at[sl, pl.ds(r0, _RPS_BASE)])

            @pl.when(sid < _NS - 1)
            def _():
                pltpu.sync_copy(
                    acc.at[pl.ds(r0 + _RPS_BASE, _RPS_EXTRA)],
                    out_hbm.at[sl, pl.ds(r0 + _RPS_BASE, _RPS_EXTRA)])

            if p + 1 < P:
                plsc.subcore_barrier()

    return edge_agg


def _edge_agg_4(m_flat, src2d, dst2d):
    return _make_edge_agg(4)(m_flat, src2d, dst2d)


def _edge_agg_8(m_flat, src2d, dst2d):
    return _make_edge_agg(8)(m_flat, src2d, dst2d)


def _msplit_body(S, h_ref, w_ref, out_ref):
    h = h_ref[...]
    for s in range(S):
        out_ref[s] = jnp.dot(h, w_ref[s], preferred_element_type=jnp.float32)


def _msplit(h, w_sliced, BN=2000):
    F = h.shape[1]
    S = F // _FW
    nblk = N // BN
    return pl.pallas_call(
        functools.partial(_msplit_body, S),
        grid=(nblk,),
        in_specs=[
            pl.BlockSpec((BN, F), lambda i: (i, 0)),
            pl.BlockSpec((S, F, _FW), lambda i: (0, 0, 0)),
        ],
        out_specs=pl.BlockSpec((S, BN, _FW), lambda i: (0, i, 0)),
        out_shape=jax.ShapeDtypeStruct((S, N, _FW), jnp.float32),
    )(h, w_sliced)


def _gru_body(S, mode, h_ref, agg_ref, wih_ref, whh_ref, bih_ref, bhh_ref,
              bng_ref, bnb_ref, bnm_ref, bnv_ref, out_ref):
    h = h_ref[...]
    aggf = jnp.concatenate([agg_ref[s] for s in range(S)], axis=-1)
    g = []
    for k in range(3):
        gi = jnp.dot(aggf, wih_ref[k], preferred_element_type=jnp.float32)
        gh = jnp.dot(h, whh_ref[k], preferred_element_type=jnp.float32)
        g.append((gi + bih_ref[k][None, :], gh + bhh_ref[k][None, :]))
    r = jax.nn.sigmoid(g[0][0] + g[0][1])
    z = jax.nn.sigmoid(g[1][0] + g[1][1])
    nn = jnp.tanh(g[2][0] + r * g[2][1])
    hn = (1.0 - z) * nn + z * h
    if mode >= 1:
        hn = _elu(hn)
    if mode == 1:
        hn = ((hn - bnm_ref[0][None, :])
              / jnp.sqrt(bnv_ref[0][None, :] + 1e-5)
              * bng_ref[0][None, :] + bnb_ref[0][None, :])
    out_ref[...] = hn


def _gru(h, agg, wihT3, whhT3, bih3, bhh3, bng, bnb, bnm, bnv, mode,
         BN=2000):
    F = h.shape[1]
    S = F // _FW
    nblk = N // BN
    return pl.pallas_call(
        functools.partial(_gru_body, S, mode),
        grid=(nblk,),
        in_specs=[
            pl.BlockSpec((BN, F), lambda i: (i, 0)),
            pl.BlockSpec((S, BN, _FW), lambda i: (0, i, 0)),
            pl.BlockSpec((3, F, F), lambda i: (0, 0, 0)),
            pl.BlockSpec((3, F, F), lambda i: (0, 0, 0)),
            pl.BlockSpec((3, F), lambda i: (0, 0)),
            pl.BlockSpec((3, F), lambda i: (0, 0)),
            pl.BlockSpec((1, F), lambda i: (0, 0)),
            pl.BlockSpec((1, F), lambda i: (0, 0)),
            pl.BlockSpec((1, F), lambda i: (0, 0)),
            pl.BlockSpec((1, F), lambda i: (0, 0)),
        ],
        out_specs=pl.BlockSpec((BN, F), lambda i: (i, 0)),
        out_shape=jax.ShapeDtypeStruct((N, F), jnp.float32),
    )(h, agg, wihT3, whhT3, bih3, bhh3, bng, bnb, bnm, bnv)


def _pool_body(batch_ref, h_ref, out_ref):
    i = pl.program_id(0)

    @pl.when(i == 0)
    def _():
        out_ref[...] = jnp.zeros_like(out_ref)

    b = batch_ref[...]
    seg = jax.lax.broadcasted_iota(jnp.int32, (b.shape[0], NG), 1)
    onehot = (b == seg).astype(jnp.float32)
    out_ref[...] += jax.lax.dot_general(
        onehot, h_ref[...], (((0,), (0,)), ((), ())),
        preferred_element_type=jnp.float32,
        precision=jax.lax.Precision.HIGHEST)


def _pool(batch2d, h, BN=2000):
    F = h.shape[1]
    nblk = N // BN
    return pl.pallas_call(
        _pool_body,
        grid=(nblk,),
        in_specs=[
            pl.BlockSpec((BN, 1), lambda i: (i, 0)),
            pl.BlockSpec((BN, F), lambda i: (i, 0)),
        ],
        out_specs=pl.BlockSpec((NG, F), lambda i: (0, 0)),
        out_shape=jax.ShapeDtypeStruct((NG, F), jnp.float32),
        compiler_params=pltpu.CompilerParams(
            dimension_semantics=("arbitrary",)),
    )(batch2d, h)


def _head_body(pooled_ref, f1w_ref, f1b_ref, f2w_ref, f2b_ref, out_ref):
    pooled = pooled_ref[...]
    h = _elu(jnp.dot(pooled, f1w_ref[...].T,
                     preferred_element_type=jnp.float32) + f1b_ref[...])
    logits = jnp.dot(h, f2w_ref[...].T,
                     preferred_element_type=jnp.float32) + f2b_ref[...]
    m = jnp.max(logits, axis=-1, keepdims=True)
    s = logits - m
    lse = jnp.log(jnp.sum(jnp.exp(s), axis=-1, keepdims=True))
    out_ref[...] = s - lse


def _head(pooled, f1w, f1b, f2w, f2b):
    return pl.pallas_call(
        _head_body,
        out_shape=jax.ShapeDtypeStruct((pooled.shape[0], f2w.shape[0]),
                                       jnp.float32),
    )(pooled, f1w, f1b.reshape(1, -1), f2w, f2b.reshape(1, -1))


def _layer(h, src2d, dst2d, conv_w, wih, whh, bih, bhh, bn_params,
           final_mode):
    F = conv_w.shape[-1]
    S = F // _FW
    steps = conv_w.shape[0]
    w_sliced = conv_w.reshape(steps, F, S, _FW).transpose(0, 2, 1, 3)
    wihT3 = wih.reshape(3, F, F).transpose(0, 2, 1)
    whhT3 = whh.reshape(3, F, F).transpose(0, 2, 1)
    bih3 = bih.reshape(3, F)
    bhh3 = bhh.reshape(3, F)
    edge_agg = _edge_agg_4 if S == 4 else _edge_agg_8
    for i in range(steps):
        m_split = _msplit(h, w_sliced[i])
        agg = edge_agg(m_split.reshape(S * N, _FW), src2d, dst2d)
        mode = final_mode if i == steps - 1 else 0
        h = _gru(h, agg, wihT3, whhT3, bih3, bhh3, *bn_params, mode)
    return h


def kernel(x, edge_index, batch, conv1_w, gru1_wih, gru1_whh, gru1_bih,
           gru1_bhh, bn1_g, bn1_b, bn1_m, bn1_v, conv2_w, gru2_wih, gru2_whh,
           gru2_bih, gru2_bhh, bn2_g, bn2_b, bn2_m, bn2_v, conv3_w, gru3_wih,
           gru3_whh, gru3_bih, gru3_bhh, fc1_w, fc1_b, fc2_w, fc2_b):
    src = edge_index[0]
    dst = edge_index[1]
    npad = _EPAD - E
    src2d = jnp.concatenate(
        [src, jnp.zeros((npad,), jnp.int32)]).reshape(_EDGE_BLOCKS, _KB)
    dst2d = jnp.concatenate(
        [dst, jnp.full((npad,), N, jnp.int32)]).reshape(_EDGE_BLOCKS, _KB)

    bn1 = tuple(a.reshape(1, -1) for a in (bn1_g, bn1_b, bn1_m, bn1_v))
    bn2 = tuple(a.reshape(1, -1) for a in (bn2_g, bn2_b, bn2_m, bn2_v))
    d128 = jnp.zeros((1, 128), jnp.float32)
    bn3 = (d128, d128, d128, d128)

    h = _layer(x, src2d, dst2d, conv1_w, gru1_wih, gru1_whh, gru1_bih,
               gru1_bhh, bn1, final_mode=1)
    h = _layer(h, src2d, dst2d, conv2_w, gru2_wih, gru2_whh, gru2_bih,
               gru2_bhh, bn2, final_mode=1)
    h = jnp.concatenate([h, jnp.zeros((N, 64), jnp.float32)], axis=1)
    h = _layer(h, src2d, dst2d, conv3_w, gru3_wih, gru3_whh, gru3_bih,
               gru3_bhh, bn3, final_mode=2)

    pooled = _pool(batch.reshape(N, 1), h)
    return _head(pooled, fc1_w, fc1_b, fc2_w, fc2_b)


# speedup vs baseline: 4.1285x; 1.0742x over previous
import functools

import jax
import jax.numpy as jnp
from jax import lax
from jax.experimental import pallas as pl
from jax.experimental.pallas import tpu as pltpu
from jax.experimental.pallas import tpu_sc as plsc

N = 50000
E = 800000
NG = 128

_NC = 2
_NS = 16
_KB = 128
_BLK_PER_CHUNK = 8
_CHUNK = _KB * _BLK_PER_CHUNK
_EDGE_BLOCKS = -(-E // (_KB * _NS * _BLK_PER_CHUNK)) * _NS * _BLK_PER_CHUNK
_EPAD = _EDGE_BLOCKS * _KB
_BLK_PER_SUB = _EDGE_BLOCKS // _NS
_CHUNKS_PER_SUB = _BLK_PER_SUB // _BLK_PER_CHUNK
_RPS = 3128
_RPS_BASE = 3080
_RPS_EXTRA = _RPS - _RPS_BASE
_ZROWS = 616
_ACC_ROWS = N + 16
_FW = 16


def _expm1(x):
    u = jnp.exp(x)
    em = jnp.where(u == 1.0, x,
                   jnp.where(u == 0.0, -1.0,
                             (u - 1.0) * x / jnp.log(jnp.maximum(u, 1e-38))))
    return em


def _elu(x):
    return jnp.where(x > 0, x, _expm1(jnp.minimum(x, 0.0)))


def _make_edge_agg(S):
    P = S // _NC
    mesh = plsc.VectorSubcoreMesh(core_axis_name="c", subcore_axis_name="s")

    @functools.partial(
        pl.kernel,
        mesh=mesh,
        out_type=jax.ShapeDtypeStruct((S, N, _FW), jnp.float32),
        scratch_types=[
            pltpu.VMEM((_ZROWS, _FW), jnp.float32),
            pltpu.VMEM((_BLK_PER_CHUNK, _KB), jnp.int32),
            pltpu.VMEM((_BLK_PER_CHUNK, _KB), jnp.int32),
            pltpu.VMEM((_BLK_PER_CHUNK, _KB), jnp.int32),
            pltpu.VMEM((_BLK_PER_CHUNK, _KB, _FW), jnp.float32),
            pltpu.VMEM_SHARED((_ACC_ROWS, _FW), jnp.float32),
            pltpu.SemaphoreType.DMA,
            pltpu.SemaphoreType.DMA,
        ],
        compiler_params=pltpu.CompilerParams(use_tc_tiling_on_sc=False),
    )
    def edge_agg(m_hbm, src_hbm, dst_hbm, out_hbm, zbuf, srcv, dstv, gidx,
                 rows, acc, gsem, ssem):
        cid = lax.axis_index("c")
        sid = lax.axis_index("s")

        z16 = jnp.zeros((16,), jnp.float32)

        def zbody(r, _):
            zbuf[r, pl.ds(0, 16)] = z16
            return 0

        lax.fori_loop(0, _ZROWS, zbody, 0)

        r0 = sid * _RPS
        for p in range(P):
            sl = cid * P + p
            off = sl * N

            for zi in range(_RPS_BASE // _ZROWS):
                pltpu.sync_copy(zbuf, acc.at[pl.ds(r0 + zi * _ZROWS, _ZROWS)])

            @pl.when(sid < _NS - 1)
            def _():
                pltpu.sync_copy(
                    zbuf.at[pl.ds(0, _RPS_EXTRA)],
                    acc.at[pl.ds(r0 + _RPS_BASE, _RPS_EXTRA)])

            plsc.subcore_barrier()

            def chunk_body(ch, _):
                blk0 = sid * _BLK_PER_SUB + ch * _BLK_PER_CHUNK
                c1 = pltpu.async_copy(
                    src_hbm.at[pl.ds(blk0, _BLK_PER_CHUNK)], srcv, gsem)
                c2 = pltpu.async_copy(
                    dst_hbm.at[pl.ds(blk0, _BLK_PER_CHUNK)], dstv, gsem)
                c1.wait()
                c2.wait()

                def add_body(r, _):
                    for t in range(_KB // 16):
                        gidx[r, pl.ds(t * 16, 16)] = (
                            srcv[r, pl.ds(t * 16, 16)] + off)
                    return 0

                lax.fori_loop(0, _BLK_PER_CHUNK, add_body, 0)

                copies = []
                for b in range(_BLK_PER_CHUNK):
                    copies.append(pltpu.async_copy(
                        m_hbm.at[gidx.at[b]], rows.at[b], gsem))
                for cp in copies:
                    cp.wait()

                scopies = []
                for b in range(_BLK_PER_CHUNK):
                    scopies.append(pltpu.async_copy(
                        rows.at[b], acc.at[dstv.at[b]], ssem, add=True))
                for cp in scopies:
                    cp.wait()
                return 0

            lax.fori_loop(0, _CHUNKS_PER_SUB, chunk_body, 0)
            plsc.subcore_barrier()

            pltpu.sync_copy(acc.at[pl.ds(r0, _RPS_BASE)],
                            out_hbm.at[sl, pl.ds(r0, _RPS_BASE)])

            @pl.when(sid < _NS - 1)
            def _():
                pltpu.sync_copy(
                    acc.at[pl.ds(r0 + _RPS_BASE, _RPS_EXTRA)],
                    out_hbm.at[sl, pl.ds(r0 + _RPS_BASE, _RPS_EXTRA)])

            if p + 1 < P:
                plsc.subcore_barrier()

    return edge_agg


def _edge_agg_4(m_flat, src2d, dst2d):
    return _make_edge_agg(4)(m_flat, src2d, dst2d)


def _edge_agg_8(m_flat, src2d, dst2d):
    return _make_edge_agg(8)(m_flat, src2d, dst2d)


def _msplit_body(S, h_ref, w_ref, out_ref):
    h = h_ref[...]
    for s in range(S):
        out_ref[s] = jnp.dot(h, w_ref[s], preferred_element_type=jnp.float32)


def _msplit(h, w_sliced, BN=2000):
    F = h.shape[1]
    S = F // _FW
    nblk = N // BN
    return pl.pallas_call(
        functools.partial(_msplit_body, S),
        grid=(nblk,),
        in_specs=[
            pl.BlockSpec((BN, F), lambda i: (i, 0)),
            pl.BlockSpec((S, F, _FW), lambda i: (0, 0, 0)),
        ],
        out_specs=pl.BlockSpec((S, BN, _FW), lambda i: (0, i, 0)),
        out_shape=jax.ShapeDtypeStruct((S, N, _FW), jnp.float32),
    )(h, w_sliced)


def _gru_body(S, mode, h_ref, agg_ref, wih_ref, whh_ref, bih_ref, bhh_ref,
              bng_ref, bnb_ref, bnm_ref, bnv_ref, out_ref):
    h = h_ref[...]
    aggf = jnp.concatenate([agg_ref[s] for s in range(S)], axis=-1)
    g = []
    for k in range(3):
        gi = jnp.dot(aggf, wih_ref[k], preferred_element_type=jnp.float32)
        gh = jnp.dot(h, whh_ref[k], preferred_element_type=jnp.float32)
        g.append((gi + bih_ref[k][None, :], gh + bhh_ref[k][None, :]))
    r = jax.nn.sigmoid(g[0][0] + g[0][1])
    z = jax.nn.sigmoid(g[1][0] + g[1][1])
    nn = jnp.tanh(g[2][0] + r * g[2][1])
    hn = (1.0 - z) * nn + z * h
    if mode >= 1:
        hn = _elu(hn)
    if mode == 1:
        hn = ((hn - bnm_ref[0][None, :])
              / jnp.sqrt(bnv_ref[0][None, :] + 1e-5)
              * bng_ref[0][None, :] + bnb_ref[0][None, :])
    out_ref[...] = hn


def _gru(h, agg, wihT3, whhT3, bih3, bhh3, bng, bnb, bnm, bnv, mode,
         BN=2000):
    F = h.shape[1]
    S = F // _FW
    nblk = N // BN
    return pl.pallas_call(
        functools.partial(_gru_body, S, mode),
        grid=(nblk,),
        in_specs=[
            pl.BlockSpec((BN, F), lambda i: (i, 0)),
            pl.BlockSpec((S, BN, _FW), lambda i: (0, i, 0)),
            pl.BlockSpec((3, F, F), lambda i: (0, 0, 0)),
            pl.BlockSpec((3, F, F), lambda i: (0, 0, 0)),
            pl.BlockSpec((3, F), lambda i: (0, 0)),
            pl.BlockSpec((3, F), lambda i: (0, 0)),
            pl.BlockSpec((1, F), lambda i: (0, 0)),
            pl.BlockSpec((1, F), lambda i: (0, 0)),
            pl.BlockSpec((1, F), lambda i: (0, 0)),
            pl.BlockSpec((1, F), lambda i: (0, 0)),
        ],
        out_specs=pl.BlockSpec((BN, F), lambda i: (i, 0)),
        out_shape=jax.ShapeDtypeStruct((N, F), jnp.float32),
    )(h, agg, wihT3, whhT3, bih3, bhh3, bng, bnb, bnm, bnv)


def _pool_body(batch_ref, h_ref, out_ref):
    i = pl.program_id(0)

    @pl.when(i == 0)
    def _():
        out_ref[...] = jnp.zeros_like(out_ref)

    b = batch_ref[...]
    seg = jax.lax.broadcasted_iota(jnp.int32, (b.shape[0], NG), 1)
    onehot = (b == seg).astype(jnp.float32)
    out_ref[...] += jax.lax.dot_general(
        onehot, h_ref[...], (((0,), (0,)), ((), ())),
        preferred_element_type=jnp.float32,
        precision=jax.lax.Precision.HIGHEST)


def _pool(batch2d, h, BN=2000):
    F = h.shape[1]
    nblk = N // BN
    return pl.pallas_call(
        _pool_body,
        grid=(nblk,),
        in_specs=[
            pl.BlockSpec((BN, 1), lambda i: (i, 0)),
            pl.BlockSpec((BN, F), lambda i: (i, 0)),
        ],
        out_specs=pl.BlockSpec((NG, F), lambda i: (0, 0)),
        out_shape=jax.ShapeDtypeStruct((NG, F), jnp.float32),
        compiler_params=pltpu.CompilerParams(
            dimension_semantics=("arbitrary",)),
    )(batch2d, h)


def _head_body(pooled_ref, f1w_ref, f1b_ref, f2w_ref, f2b_ref, out_ref):
    pooled = pooled_ref[...]
    h = _elu(jnp.dot(pooled, f1w_ref[...].T,
                     preferred_element_type=jnp.float32) + f1b_ref[...])
    logits = jnp.dot(h, f2w_ref[...].T,
                     preferred_element_type=jnp.float32) + f2b_ref[...]
    m = jnp.max(logits, axis=-1, keepdims=True)
    s = logits - m
    lse = jnp.log(jnp.sum(jnp.exp(s), axis=-1, keepdims=True))
    out_ref[...] = s - lse


def _head(pooled, f1w, f1b, f2w, f2b):
    return pl.pallas_call(
        _head_body,
        out_shape=jax.ShapeDtypeStruct((pooled.shape[0], f2w.shape[0]),
                                       jnp.float32),
    )(pooled, f1w, f1b.reshape(1, -1), f2w, f2b.reshape(1, -1))


def _layer(h, src2d, dst2d, conv_w, wih, whh, bih, bhh, bn_params,
           final_mode):
    F = conv_w.shape[-1]
    S = F // _FW
    steps = conv_w.shape[0]
    w_sliced = conv_w.reshape(steps, F, S, _FW).transpose(0, 2, 1, 3)
    wihT3 = wih.reshape(3, F, F).transpose(0, 2, 1)
    whhT3 = whh.reshape(3, F, F).transpose(0, 2, 1)
    bih3 = bih.reshape(3, F)
    bhh3 = bhh.reshape(3, F)
    edge_agg = _edge_agg_4 if S == 4 else _edge_agg_8
    for i in range(steps):
        m_split = _msplit(h, w_sliced[i])
        agg = edge_agg(m_split.reshape(S * N, _FW), src2d, dst2d)
        mode = final_mode if i == steps - 1 else 0
        h = _gru(h, agg, wihT3, whhT3, bih3, bhh3, *bn_params, mode)
    return h


def kernel(x, edge_index, batch, conv1_w, gru1_wih, gru1_whh, gru1_bih,
           gru1_bhh, bn1_g, bn1_b, bn1_m, bn1_v, conv2_w, gru2_wih, gru2_whh,
           gru2_bih, gru2_bhh, bn2_g, bn2_b, bn2_m, bn2_v, conv3_w, gru3_wih,
           gru3_whh, gru3_bih, gru3_bhh, fc1_w, fc1_b, fc2_w, fc2_b):
    src = edge_index[0]
    dst = edge_index[1]
    npad = _EPAD - E
    src2d = jnp.concatenate(
        [src, jnp.zeros((npad,), jnp.int32)]).reshape(_EDGE_BLOCKS, _KB)
    dst2d = jnp.concatenate(
        [dst, jnp.full((npad,), N, jnp.int32)]).reshape(_EDGE_BLOCKS, _KB)

    bn1 = tuple(a.reshape(1, -1) for a in (bn1_g, bn1_b, bn1_m, bn1_v))
    bn2 = tuple(a.reshape(1, -1) for a in (bn2_g, bn2_b, bn2_m, bn2_v))
    d128 = jnp.zeros((1, 128), jnp.float32)
    bn3 = (d128, d128, d128, d128)

    h = _layer(x, src2d, dst2d, conv1_w, gru1_wih, gru1_whh, gru1_bih,
               gru1_bhh, bn1, final_mode=1)
    h = _layer(h, src2d, dst2d, conv2_w, gru2_wih, gru2_whh, gru2_bih,
               gru2_bhh, bn2, final_mode=1)
    h = jnp.concatenate([h, jnp.zeros((N, 64), jnp.float32)], axis=1)
    h = _layer(h, src2d, dst2d, conv3_w, gru3_wih, gru3_whh, gru3_bih,
               gru3_bhh, bn3, final_mode=2)

    pooled = _pool(batch.reshape(N, 1), h)
    return _head(pooled, fc1_w, fc1_b, fc2_w, fc2_b)
